# baseline (device time: 13345 ns/iter reference)
import jax
import jax.numpy as jnp
from jax import lax
from jax.experimental import pallas as pl
from jax.experimental.pallas import tpu as pltpu

N_CHUNKS = 4

_SEM_XA = 0
_SEM_XB0 = 1
_SEM_YF = 1 + N_CHUNKS
_N_SEMS = 2 + N_CHUNKS


def kernel(A, B):
    m, k = A.shape
    _, n = B.shape
    nc = n // N_CHUNKS
    mh = m // 2

    def body(a_ref, b_ref, out_ref, a_bf, b_bf, a_rcv, b_rcv,
             send_sems, recv_sems):
        my_x = lax.axis_index("x")
        my_y = lax.axis_index("y")
        xp = (1 - my_x, my_y)
        yp = (my_x, 1 - my_y)

        mine = pl.ds(my_y * mh, mh)
        other = pl.ds((1 - my_y) * mh, mh)

        a_bf[...] = a_ref[...].astype(jnp.bfloat16)
        for j in range(N_CHUNKS):
            b_bf[j] = b_ref[:, pl.ds(j * nc, nc)].astype(jnp.bfloat16)

        barrier_sem = pltpu.get_barrier_semaphore()
        for nbr in (xp, yp):
            pl.semaphore_signal(
                barrier_sem, inc=1,
                device_id=nbr, device_id_type=pl.DeviceIdType.MESH,
            )
        pl.semaphore_wait(barrier_sem, 2)

        rdma_xa = pltpu.make_async_remote_copy(
            src_ref=a_bf.at[mine], dst_ref=a_rcv.at[mine],
            send_sem=send_sems.at[_SEM_XA], recv_sem=recv_sems.at[_SEM_XA],
            device_id=xp, device_id_type=pl.DeviceIdType.MESH,
        )
        rdma_xa.start()
        rdma_bs = []
        for j in range(N_CHUNKS):
            r = pltpu.make_async_remote_copy(
                src_ref=b_bf.at[j], dst_ref=b_rcv.at[j],
                send_sem=send_sems.at[_SEM_XB0 + j],
                recv_sem=recv_sems.at[_SEM_XB0 + j],
                device_id=xp, device_id_type=pl.DeviceIdType.MESH,
            )
            r.start()
            rdma_bs.append(r)

        for j in range(N_CHUNKS):
            out_ref[:, pl.ds(j * nc, nc)] = jnp.dot(
                a_bf[...], b_bf[j], preferred_element_type=jnp.float32
            )

        rdma_xa.wait_recv()
        rdma_yf = pltpu.make_async_remote_copy(
            src_ref=a_rcv.at[mine], dst_ref=a_rcv.at[mine],
            send_sem=send_sems.at[_SEM_YF], recv_sem=recv_sems.at[_SEM_YF],
            device_id=yp, device_id_type=pl.DeviceIdType.MESH,
        )
        rdma_yf.start()

        fwd_in = pltpu.make_async_remote_copy(
            src_ref=a_rcv.at[other], dst_ref=a_rcv.at[other],
            send_sem=send_sems.at[_SEM_YF], recv_sem=recv_sems.at[_SEM_YF],
            device_id=yp, device_id_type=pl.DeviceIdType.MESH,
        )
        fwd_in.wait_recv()

        for j in range(N_CHUNKS):
            rdma_bs[j].wait_recv()
            out_ref[:, pl.ds(j * nc, nc)] += jnp.dot(
                a_rcv[...], b_rcv[j], preferred_element_type=jnp.float32
            )

        rdma_xa.wait_send()
        rdma_yf.wait_send()
        for j in range(N_CHUNKS):
            rdma_bs[j].wait_send()

    return pl.pallas_call(
        body,
        out_shape=jax.ShapeDtypeStruct((m, n), jnp.float32),
        in_specs=[
            pl.BlockSpec(memory_space=pltpu.MemorySpace.VMEM),
            pl.BlockSpec(memory_space=pltpu.MemorySpace.VMEM),
        ],
        out_specs=pl.BlockSpec(memory_space=pltpu.MemorySpace.VMEM),
        scratch_shapes=[
            pltpu.VMEM((m, k), jnp.bfloat16),
            pltpu.VMEM((N_CHUNKS, k, nc), jnp.bfloat16),
            pltpu.VMEM((m, k), jnp.bfloat16),
            pltpu.VMEM((N_CHUNKS, k, nc), jnp.bfloat16),
            pltpu.SemaphoreType.DMA((_N_SEMS,)),
            pltpu.SemaphoreType.DMA((_N_SEMS,)),
        ],
        compiler_params=pltpu.CompilerParams(collective_id=0),
    )(A, B)
